# baseline (speedup 1.0000x reference)
import functools

import jax
import jax.numpy as jnp
from jax import lax
from jax.experimental import pallas as pl
from jax.experimental.pallas import tpu as pltpu
from jax.experimental.pallas import tpu_sc as plsc

_LANES = 16


def _build_sc_kernel(B, S, D):
    mesh = plsc.VectorSubcoreMesh(
        core_axis_name="c", subcore_axis_name="s", num_cores=1
    )
    half = S // 2

    @functools.partial(
        pl.kernel,
        mesh=mesh,
        out_type=jax.ShapeDtypeStruct((B, D), jnp.float32),
        scratch_types=[
            pltpu.VMEM((S,), jnp.float32),
            pltpu.SemaphoreType.DMA,
            pltpu.SemaphoreType.DMA,
        ],
    )
    def k(do_hbm, mask_hbm, out_hbm, mask_v, sem0, sem1):
        b = lax.axis_index("s")

        @pl.when(b < B)
        def _():
            cp0 = pltpu.async_copy(
                mask_hbm.at[b, pl.ds(0, half)], mask_v.at[pl.ds(0, half)], sem0
            )
            cp1 = pltpu.async_copy(
                mask_hbm.at[b, pl.ds(half, half)],
                mask_v.at[pl.ds(half, half)],
                sem1,
            )

            unroll = 16
            span = unroll * _LANES

            def mk_body(offset):
                def body(i, acc):
                    base = offset + i * span
                    for j in range(unroll):
                        acc = acc + mask_v[pl.ds(base + j * _LANES, _LANES)]
                    return acc

                return body

            cp0.wait()
            acc = lax.fori_loop(
                0, half // span, mk_body(0), jnp.zeros((_LANES,), jnp.float32)
            )
            cp1.wait()
            acc = lax.fori_loop(0, half // span, mk_body(half), acc)

            acc_i = acc.astype(jnp.int32)
            lanes = [acc_i[j] for j in range(_LANES)]
            while len(lanes) > 1:
                lanes = [
                    lanes[j] + lanes[j + 1] for j in range(0, len(lanes), 2)
                ]
            total = lanes[0]

            idx = jnp.maximum(total - 1, 0)
            row = b * S + idx
            pltpu.sync_copy(do_hbm.at[pl.ds(row, 1)], out_hbm.at[pl.ds(b, 1)])

    return k


@jax.jit
def kernel(decoder_outputs, attention_mask):
    B, S, D = decoder_outputs.shape
    do2d = decoder_outputs.reshape(B * S, D)
    k = _build_sc_kernel(B, S, D)
    return k(do2d, attention_mask.astype(jnp.float32))


# speedup vs baseline: 1.0539x; 1.0539x over previous
import functools

import jax
import jax.numpy as jnp
from jax import lax
from jax.experimental import pallas as pl
from jax.experimental.pallas import tpu as pltpu
from jax.experimental.pallas import tpu_sc as plsc

_LANES = 16


def _build_sc_kernel(B, S, D):
    mesh = plsc.VectorSubcoreMesh(
        core_axis_name="c", subcore_axis_name="s", num_cores=1
    )
    num_cores = 1

    @functools.partial(
        pl.kernel,
        mesh=mesh,
        out_type=jax.ShapeDtypeStruct((B, D), jnp.float32),
        scratch_types=[
            pltpu.VMEM((S,), jnp.float32),
            pltpu.VMEM((1, D), jnp.float32),
        ],
    )
    def k(do_hbm, mask_hbm, out_hbm, mask_v, row_v):
        w = lax.axis_index("s") * num_cores + lax.axis_index("c")

        @pl.when(w < B)
        def _():
            pltpu.sync_copy(mask_hbm.at[w], mask_v)

            unroll = 16
            span = unroll * _LANES

            def body(i, acc):
                base = i * span
                for j in range(unroll):
                    acc = acc + mask_v[pl.ds(base + j * _LANES, _LANES)]
                return acc

            acc = lax.fori_loop(
                0, S // span, body, jnp.zeros((_LANES,), jnp.float32)
            )
            acc_i = acc.astype(jnp.int32)
            lanes = [acc_i[j] for j in range(_LANES)]
            while len(lanes) > 1:
                lanes = [
                    lanes[j] + lanes[j + 1] for j in range(0, len(lanes), 2)
                ]
            total = lanes[0]
            idx = jnp.maximum(total - 1, 0)
            row = w * S + idx
            pltpu.sync_copy(do_hbm.at[pl.ds(row, 1)], row_v)
            pltpu.sync_copy(row_v, out_hbm.at[pl.ds(w, 1)])

    return k


@jax.jit
def kernel(decoder_outputs, attention_mask):
    B, S, D = decoder_outputs.shape
    do2d = decoder_outputs.reshape(B * S, D)
    k = _build_sc_kernel(B, S, D)
    return k(do2d, attention_mask.astype(jnp.float32))


# speedup vs baseline: 1.0609x; 1.0066x over previous
import functools

import jax
import jax.numpy as jnp
from jax import lax
from jax.experimental import pallas as pl
from jax.experimental.pallas import tpu as pltpu
from jax.experimental.pallas import tpu_sc as plsc

_LANES = 16


def _build_sc_kernel(B, S, D):
    mesh = plsc.VectorSubcoreMesh(
        core_axis_name="c", subcore_axis_name="s", num_cores=1
    )
    num_cores = 1

    @functools.partial(
        pl.kernel,
        mesh=mesh,
        out_type=jax.ShapeDtypeStruct((B, D), jnp.float32),
        scratch_types=[
            pltpu.VMEM((S,), jnp.float32),
            pltpu.VMEM((1, D), jnp.float32),
            pltpu.SemaphoreType.DMA,
            pltpu.SemaphoreType.DMA,
        ],
    )
    def k(do_hbm, mask_hbm, out_hbm, mask_v, row_v, sem0, sem1):
        w = lax.axis_index("s") * num_cores + lax.axis_index("c")

        @pl.when(w < B)
        def _():
            half = S // 2
            cp0 = pltpu.async_copy(
                mask_hbm.at[w, pl.ds(0, half)],
                mask_v.at[pl.ds(0, half)],
                sem0,
            )
            cp1 = pltpu.async_copy(
                mask_hbm.at[w, pl.ds(half, half)],
                mask_v.at[pl.ds(half, half)],
                sem1,
            )

            unroll = 16
            span = unroll * _LANES

            def body(i, acc):
                base = i * span
                for j in range(unroll):
                    acc = acc + mask_v[pl.ds(base + j * _LANES, _LANES)]
                return acc

            cp0.wait()
            acc = lax.fori_loop(
                0, half // span, body, jnp.zeros((_LANES,), jnp.float32)
            )
            cp1.wait()

            def body2(i, acc):
                base = half + i * span
                for j in range(unroll):
                    acc = acc + mask_v[pl.ds(base + j * _LANES, _LANES)]
                return acc

            acc = lax.fori_loop(0, half // span, body2, acc)
            acc_i = acc.astype(jnp.int32)
            lanes = [acc_i[j] for j in range(_LANES)]
            while len(lanes) > 1:
                lanes = [
                    lanes[j] + lanes[j + 1] for j in range(0, len(lanes), 2)
                ]
            total = lanes[0]
            idx = jnp.maximum(total - 1, 0)
            row = w * S + idx
            pltpu.sync_copy(do_hbm.at[pl.ds(row, 1)], row_v)
            pltpu.sync_copy(row_v, out_hbm.at[pl.ds(w, 1)])

    return k


@jax.jit
def kernel(decoder_outputs, attention_mask):
    B, S, D = decoder_outputs.shape
    do2d = decoder_outputs.reshape(B * S, D)
    k = _build_sc_kernel(B, S, D)
    return k(do2d, attention_mask.astype(jnp.float32))
